# baseline (speedup 1.0000x reference)
import functools

import jax
import jax.numpy as jnp
from jax import lax
from jax.experimental import pallas as pl
from jax.experimental.pallas import tpu as pltpu
from jax.experimental.pallas import tpu_sc as plsc

_N = 10000
_G = 64
_D = 48
_NW = 32
_CHUNK = 384
_P = 64
_AROWS = _G + 2
_AFLAT = _AROWS * _P

_sc_mesh = plsc.VectorSubcoreMesh(core_axis_name="c", subcore_axis_name="s")


@functools.partial(
    pl.kernel,
    out_type=jax.ShapeDtypeStruct((_NW, _AROWS, _P), jnp.float32),
    mesh=_sc_mesh,
    scratch_types=[
        pltpu.VMEM((_D, _CHUNK), jnp.float32),
        pltpu.VMEM((_D * _CHUNK,), jnp.float32),
        pltpu.VMEM((_N + 16,), jnp.int32),
        pltpu.VMEM((_AROWS, _P), jnp.float32),
    ],
    compiler_params=pltpu.CompilerParams(needs_layout_passes=False),
)
def _sc_segsum(xt_hbm, b_hbm, out_hbm, x_v, xu_v, b_v, acc_v):
    wid = lax.axis_index("c") * 16 + lax.axis_index("s")
    base = 128 * jnp.minimum(3 * wid, 76)
    end = jnp.minimum(_CHUNK, _N - base)
    start = jnp.clip(_CHUNK * wid - base, 0, end)

    pltpu.sync_copy(xt_hbm.at[:, pl.ds(base, _CHUNK)], x_v)
    pltpu.sync_copy(b_hbm, b_v.at[pl.ds(0, _N)])

    lane = lax.iota(jnp.int32, 16)
    zeros16 = jnp.zeros((16,), jnp.float32)
    one0 = jnp.where(lane == 0, 1.0, 0.0).astype(jnp.float32)
    d0 = lane
    d1 = lane + 16
    d2 = lane + 32

    def _zero(j, _):
        for k in range(_P // 16):
            acc_v[j, pl.ds(16 * k, 16)] = zeros16
        return 0
    lax.fori_loop(0, _AROWS, _zero, 0)

    def _retile_d(d, _):
        def _retile_c(cc, _2):
            v = x_v[d, pl.ds(pl.multiple_of(cc * 16, 16), 16)]
            xu_v[pl.ds(d * _CHUNK + cc * 16, 16)] = v
            return 0
        lax.fori_loop(0, _CHUNK // 16, _retile_c, 0)
        return 0
    lax.fori_loop(0, _D, _retile_d, 0)

    def _row(r, carry):
        sq0, sq1, sq2 = carry
        g = b_v[pl.ds(base + r, 16)][0]
        g_vec = lane * 0 + g
        col = lane * 0 + r
        v0 = plsc.load_gather(xu_v, [d0 * _CHUNK + col])
        v1 = plsc.load_gather(xu_v, [d1 * _CHUNK + col])
        v2 = plsc.load_gather(xu_v, [d2 * _CHUNK + col])
        plsc.addupdate_scatter(acc_v, [g_vec, d0], v0)
        plsc.addupdate_scatter(acc_v, [g_vec, d1], v1)
        plsc.addupdate_scatter(acc_v, [g_vec, d2], v2)
        plsc.addupdate_scatter(acc_v, [g_vec, lane + 48], one0)
        return (sq0 + v0 * v0, sq1 + v1 * v1, sq2 + v2 * v2)

    sq0, sq1, sq2 = lax.fori_loop(
        start, end, _row, (zeros16, zeros16, zeros16))

    acc_v[_G + 1, pl.ds(0, 16)] = sq0
    acc_v[_G + 1, pl.ds(16, 16)] = sq1
    acc_v[_G + 1, pl.ds(32, 16)] = sq2

    pltpu.sync_copy(acc_v, out_hbm.at[wid])


def _tc_final_body(acc_ref, g_ref, be_ref, w2a_ref, b2a_ref, w2b_ref,
                   b2b_ref, w2c_ref, b2c_ref, w2d_ref, b2d_ref, wo_row_ref,
                   bo_ref, out_ref):
    acc = jnp.sum(acc_ref[:, :, :], axis=0)
    n = jnp.float32(_N)

    sums = acc[:_G, :_D]
    cnt = acc[:_G, _D:_D + 1]
    colsq = acc[_G + 1:_G + 2, :_D]
    colsum = jnp.sum(sums, axis=0, keepdims=True)
    mu = colsum / n
    var = colsq / n - mu * mu
    inv = jax.lax.rsqrt(var + 1e-5)

    gamma = g_ref[:]
    beta = be_ref[:]
    seg_bn = (sums - cnt * mu) * (inv * gamma) + cnt * beta
    x3 = seg_bn / jnp.maximum(cnt, 1.0)

    def mm(a, w_ref, bias_ref):
        return jax.lax.dot_general(
            a, w_ref[:, :], (((1,), (0,)), ((), ())),
            preferred_element_type=jnp.float32) + bias_ref[:]

    h = jnp.maximum(mm(x3, w2a_ref, b2a_ref), 0.0)
    h = jnp.maximum(mm(h, w2b_ref, b2b_ref), 0.0)
    h = jnp.maximum(mm(h, w2c_ref, b2c_ref), 0.0)
    h = mm(h, w2d_ref, b2d_ref)
    out_t = jax.lax.dot_general(h, wo_row_ref[:, :], (((1,), (1,)), ((), ())),
                                preferred_element_type=jnp.float32)
    out_ref[:, :] = out_t.reshape(1, _G) + bo_ref[:]


@jax.jit
def _fused(XT, batch, bn_gamma, bn_beta, W2a, b2a, W2b, b2b, W2c, b2c,
           W2d, b2d, Wo_row, bo):
    acc = _sc_segsum(XT, batch)
    out_t = pl.pallas_call(
        _tc_final_body,
        out_shape=jax.ShapeDtypeStruct((1, _G), jnp.float32),
    )(acc, bn_gamma, bn_beta, W2a, b2a, W2b, b2b, W2c, b2c,
      W2d, b2d, Wo_row, bo)
    return out_t.reshape(_G, 1)


def kernel(X, edge_index, batch, bn_gamma, bn_beta, W1a, b1a, W1b, b1b,
           W1c, b1c, Wc1, bc1, cn_gamma, cn_beta, Wc2, bc2, t,
           W2a, b2a, W2b, b2b, W2c, b2c, W2d, b2d, Wo, bo):
    return _fused(
        X.T,
        batch,
        bn_gamma, bn_beta,
        W2a, b2a, W2b, b2b, W2c, b2c, W2d, b2d, Wo.T, bo,
    )


# speedup vs baseline: 10.8618x; 10.8618x over previous
import jax
import jax.numpy as jnp
from jax.experimental import pallas as pl

_N = 10000
_G = 64
_D = 48


def _fused_body(xt_ref, b_ref, g_ref, be_ref, w2a_ref, b2a_ref, w2b_ref,
                b2b_ref, w2c_ref, b2c_ref, w2d_ref, b2d_ref, wo_row_ref,
                bo_ref, out_ref):
    xt = xt_ref[:, :]
    n = jnp.float32(_N)

    colsum = jnp.sum(xt, axis=1)
    colsq = jnp.sum(xt * xt, axis=1)
    mu = colsum / n
    var = colsq / n - mu * mu
    inv = jax.lax.rsqrt(var + 1e-5)

    bat = b_ref[:].reshape(1, _N)
    onehot_t = (bat == jax.lax.broadcasted_iota(jnp.int32, (_G, 1), 0)
                ).astype(jnp.float32)
    sums = jax.lax.dot_general(onehot_t, xt, (((1,), (1,)), ((), ())),
                               preferred_element_type=jnp.float32)
    cnt = jnp.sum(onehot_t, axis=1, keepdims=True)

    gamma = g_ref[:]
    beta = be_ref[:]
    seg_bn = (sums - cnt * mu) * (inv * gamma) + cnt * beta
    x3 = seg_bn / jnp.maximum(cnt, 1.0)

    def mm(a, w_ref, bias_ref):
        return jax.lax.dot_general(
            a, w_ref[:, :], (((1,), (0,)), ((), ())),
            preferred_element_type=jnp.float32) + bias_ref[:]

    h = jnp.maximum(mm(x3, w2a_ref, b2a_ref), 0.0)
    h = jnp.maximum(mm(h, w2b_ref, b2b_ref), 0.0)
    h = jnp.maximum(mm(h, w2c_ref, b2c_ref), 0.0)
    h = mm(h, w2d_ref, b2d_ref)
    out_t = jax.lax.dot_general(h, wo_row_ref[:, :], (((1,), (1,)), ((), ())),
                                preferred_element_type=jnp.float32)
    out_ref[:, :] = out_t.reshape(1, _G) + bo_ref[:]


@jax.jit
def _fused(XT, batch, bn_gamma, bn_beta, W2a, b2a, W2b, b2b, W2c, b2c,
           W2d, b2d, Wo_row, bo):
    out_t = pl.pallas_call(
        _fused_body,
        out_shape=jax.ShapeDtypeStruct((1, _G), jnp.float32),
    )(XT, batch, bn_gamma, bn_beta, W2a, b2a, W2b, b2b, W2c, b2c,
      W2d, b2d, Wo_row, bo)
    return out_t.reshape(_G, 1)


def kernel(X, edge_index, batch, bn_gamma, bn_beta, W1a, b1a, W1b, b1b,
           W1c, b1c, Wc1, bc1, cn_gamma, cn_beta, Wc2, bc2, t,
           W2a, b2a, W2b, b2b, W2c, b2c, W2d, b2d, Wo, bo):
    return _fused(
        X.T,
        batch,
        bn_gamma, bn_beta,
        W2a, b2a, W2b, b2b, W2c, b2c, W2d, b2d, Wo.T, bo,
    )
